# baseline (speedup 1.0000x reference)
import functools

import jax
import jax.numpy as jnp
from jax import lax
from jax.experimental import pallas as pl
from jax.experimental.pallas import tpu as pltpu
from jax.experimental.pallas import tpu_sc as plsc

N = 100000
D = 128
R = 14
E = 40000
L = 4
RE = R * E

NC, NS = 2, 16
NW = NC * NS

NP = 102400
BN = 2048
BE = 2000

GB = 128
TOTP = 561152
GBLK = TOTP // NW // GB

ET = TOTP // NS
UB = 2192
NUB = ET // UB
LROWS = 288

NCH = 10
KPC = NCH // NC
CH = NP // NCH
STRIPE = CH // NS
TRASH = CH
SPR = CH + 24

_BIG = 1 << 30

_mesh = plsc.VectorSubcoreMesh(core_axis_name="c", subcore_axis_name="s")


def _mm_body(x_ref, w_ref, o_ref):
    o_ref[...] = lax.dot_general(
        x_ref[...], w_ref[...], (((1,), (1,)), ((), ())),
        preferred_element_type=jnp.float32)


def _tc_mm(x, w):
    grid = (NP // BN,)
    return pl.pallas_call(
        _mm_body,
        grid=grid,
        in_specs=[
            pl.BlockSpec((BN, D), lambda i: (i, 0)),
            pl.BlockSpec((D, D), lambda i: (0, 0)),
        ],
        out_specs=pl.BlockSpec((BN, D), lambda i: (i, 0)),
        out_shape=jax.ShapeDtypeStruct((NP, D), jnp.float32),
    )(x, w)


def _msg_body(x_ref, w_ref, o_ref):
    o_ref[...] = lax.dot_general(
        x_ref[...], w_ref[0], (((1,), (1,)), ((), ())),
        preferred_element_type=jnp.float32)


def _tc_msg_mm(gath, w_rel_i):
    nb = E // BE
    grid = (R, nb)
    return pl.pallas_call(
        _msg_body,
        grid=grid,
        in_specs=[
            pl.BlockSpec((BE, D), lambda r, j: (r * nb + j, 0)),
            pl.BlockSpec((1, D, D), lambda r, j: (r, 0, 0)),
        ],
        out_specs=pl.BlockSpec((BE, D), lambda r, j: (r * nb + j, 0)),
        out_shape=jax.ShapeDtypeStruct((RE, D), jnp.float32),
    )(gath, w_rel_i)


def _post_body(t_ref, res_ref, g1w_ref, g1b_ref, w2_ref, g2w_ref, g2b_ref,
               o_ref):
    x = t_ref[...]
    mu = jnp.mean(x, axis=1, keepdims=True)
    var = jnp.mean((x - mu) ** 2, axis=1, keepdims=True)
    x = (x - mu) * lax.rsqrt(var + 1e-5) * g1w_ref[...] + g1b_ref[...]
    x = jnp.maximum(x, 0.0)
    y = lax.dot_general(x, w2_ref[...], (((1,), (1,)), ((), ())),
                        preferred_element_type=jnp.float32)
    mu2 = jnp.mean(y, axis=1, keepdims=True)
    var2 = jnp.mean((y - mu2) ** 2, axis=1, keepdims=True)
    y = (y - mu2) * lax.rsqrt(var2 + 1e-5) * g2w_ref[...] + g2b_ref[...]
    o_ref[...] = jnp.maximum(y + res_ref[...], 0.0)


def _tc_post(temp, res, g1w, g1b, w2, g2w, g2b):
    grid = (NP // BN,)
    vec = lambda i: (0, 0)
    return pl.pallas_call(
        _post_body,
        grid=grid,
        in_specs=[
            pl.BlockSpec((BN, D), lambda i: (i, 0)),
            pl.BlockSpec((BN, D), lambda i: (i, 0)),
            pl.BlockSpec((1, D), vec),
            pl.BlockSpec((1, D), vec),
            pl.BlockSpec((D, D), vec),
            pl.BlockSpec((1, D), vec),
            pl.BlockSpec((1, D), vec),
        ],
        out_specs=pl.BlockSpec((BN, D), lambda i: (i, 0)),
        out_shape=jax.ShapeDtypeStruct((NP, D), jnp.float32),
    )(temp, res, g1w, g1b, w2, g2w, g2b)


def _post_mm_body(t_ref, res_ref, g1w_ref, g1b_ref, w2_ref, g2w_ref,
                  g2b_ref, wn_ref, o_ref, o2_ref):
    _post_body(t_ref, res_ref, g1w_ref, g1b_ref, w2_ref, g2w_ref, g2b_ref,
               o_ref)
    o2_ref[...] = lax.dot_general(
        o_ref[...], wn_ref[...], (((1,), (1,)), ((), ())),
        preferred_element_type=jnp.float32)


def _tc_post_mm(temp, res, g1w, g1b, w2, g2w, g2b, w_next):
    grid = (NP // BN,)
    vec = lambda i: (0, 0)
    return pl.pallas_call(
        _post_mm_body,
        grid=grid,
        in_specs=[
            pl.BlockSpec((BN, D), lambda i: (i, 0)),
            pl.BlockSpec((BN, D), lambda i: (i, 0)),
            pl.BlockSpec((1, D), vec),
            pl.BlockSpec((1, D), vec),
            pl.BlockSpec((D, D), vec),
            pl.BlockSpec((1, D), vec),
            pl.BlockSpec((1, D), vec),
            pl.BlockSpec((D, D), vec),
        ],
        out_specs=[
            pl.BlockSpec((BN, D), lambda i: (i, 0)),
            pl.BlockSpec((BN, D), lambda i: (i, 0)),
        ],
        out_shape=[
            jax.ShapeDtypeStruct((NP, D), jnp.float32),
            jax.ShapeDtypeStruct((NP, D), jnp.float32),
        ],
    )(temp, res, g1w, g1b, w2, g2w, g2b, w_next)


@functools.partial(
    pl.kernel,
    out_type=jax.ShapeDtypeStruct((TOTP, D), jnp.float32),
    mesh=_mesh,
    scratch_types=[
        pltpu.VMEM((GBLK, GB), jnp.int32),
        pltpu.VMEM((GB, D), jnp.float32),
        pltpu.VMEM((GB, D), jnp.float32),
        pltpu.SemaphoreType.DMA,
        pltpu.SemaphoreType.DMA,
    ],
)
def _sc_gather(feat_hbm, v2d_hbm, out_hbm, idx_v, rows_a, rows_b,
               sem_a, sem_b):
    cid = lax.axis_index("c")
    sid = lax.axis_index("s")
    wid = sid * NC + cid
    pltpu.sync_copy(v2d_hbm.at[wid], idx_v)
    base_row = wid * GBLK * GB

    pltpu.async_copy(feat_hbm.at[idx_v.at[0]], rows_a, sem_a)

    def pair(q, c):
        b0 = 2 * q
        pltpu.async_copy(feat_hbm.at[idx_v.at[b0 + 1]], rows_b, sem_b)
        pltpu.make_async_copy(feat_hbm.at[idx_v.at[b0]], rows_a, sem_a).wait()
        pltpu.sync_copy(rows_a, out_hbm.at[pl.ds(base_row + b0 * GB, GB)])
        pltpu.async_copy(feat_hbm.at[idx_v.at[b0 + 2]], rows_a, sem_a)
        pltpu.make_async_copy(feat_hbm.at[idx_v.at[b0 + 1]], rows_b,
                              sem_b).wait()
        pltpu.sync_copy(rows_b,
                        out_hbm.at[pl.ds(base_row + (b0 + 1) * GB, GB)])
        return c

    lax.fori_loop(0, (GBLK - 1) // 2, pair, 0)
    pltpu.make_async_copy(feat_hbm.at[idx_v.at[GBLK - 1]], rows_a,
                          sem_a).wait()
    pltpu.sync_copy(rows_a,
                    out_hbm.at[pl.ds(base_row + (GBLK - 1) * GB, GB)])


@functools.partial(
    pl.kernel,
    out_type=(
        jax.ShapeDtypeStruct((NC, NS, KPC, LROWS, GB), jnp.int32),
        jax.ShapeDtypeStruct((NC, NS, KPC, LROWS, GB), jnp.int32),
        jax.ShapeDtypeStruct((NC, NS, 16), jnp.int32),
    ),
    mesh=_mesh,
    scratch_types=[
        pltpu.VMEM((UB,), jnp.int32),
        pltpu.VMEM((LROWS, GB), jnp.int32),
        pltpu.VMEM((LROWS, GB), jnp.int32),
        pltpu.VMEM((16,), jnp.int32),
    ],
    compiler_params=pltpu.CompilerParams(needs_layout_passes=False),
)
def _sc_build(u_hbm, eids_hbm, lidx_hbm, cnts_hbm, ubuf, eids_v, lidx_v,
              cnts_v):
    cid = lax.axis_index("c")
    sid = lax.axis_index("s")
    e0 = sid * ET
    iota = lax.iota(jnp.int32, 16)
    cnts = jnp.zeros((16,), jnp.int32)
    for k in range(KPC):
        base = (2 * k + cid) * CH
        cnt = jnp.int32(0)
        for ub in range(NUB):
            pltpu.sync_copy(u_hbm.at[pl.ds(e0 + ub * UB, UB)], ubuf)

            def scan_body(j, cnt, _ub=ub):
                u16 = ubuf[pl.ds(j * 16, 16)]
                m = (u16 >= base) & (u16 < base + CH)
                mi = jnp.where(m, 1, 0)
                pos = cnt + plsc.cumsum(mi) - 1
                eid16 = e0 + _ub * UB + j * 16 + iota
                plsc.store_scatter(eids_v, [pos >> 7, pos & 127], eid16,
                                   mask=m)
                plsc.store_scatter(lidx_v, [pos >> 7, pos & 127], u16 - base,
                                   mask=m)
                return cnt + jnp.sum(mi)

            cnt = lax.fori_loop(0, UB // 16, scan_body, cnt)
        for pb in range(2 * GB // 16):
            pos = cnt + pb * 16 + iota
            plsc.store_scatter(eids_v, [pos >> 7, pos & 127],
                               jnp.zeros((16,), jnp.int32))
            plsc.store_scatter(lidx_v, [pos >> 7, pos & 127],
                               TRASH + iota)
        pltpu.sync_copy(eids_v, eids_hbm.at[cid, sid, k])
        pltpu.sync_copy(lidx_v, lidx_hbm.at[cid, sid, k])
        cnts = jnp.where(iota == k, cnt, cnts)
    cnts_v[...] = cnts
    pltpu.sync_copy(cnts_v, cnts_hbm.at[cid, sid])


@functools.partial(
    pl.kernel,
    out_type=jax.ShapeDtypeStruct((NP, D), jnp.float32),
    mesh=_mesh,
    scratch_types=[
        pltpu.VMEM_SHARED((SPR, D), jnp.float32),
        pltpu.VMEM((16, GB), jnp.int32),
        pltpu.VMEM((16, GB), jnp.int32),
        pltpu.VMEM((GB, D), jnp.float32),
        pltpu.VMEM((GB, D), jnp.float32),
        pltpu.VMEM((16,), jnp.int32),
        pltpu.SemaphoreType.DMA,
        pltpu.SemaphoreType.DMA,
        pltpu.SemaphoreType.DMA,
        pltpu.SemaphoreType.DMA,
    ],
    compiler_params=pltpu.CompilerParams(needs_layout_passes=False),
)
def _sc_scatter(temp_hbm, msg_hbm, eids_hbm, lidx_hbm, cnts_hbm, out_hbm,
                shared, eids_s, lidx_s, rows_a, rows_b, cnts_v,
                gsem_a, gsem_b, asem_a, asem_b):
    cid = lax.axis_index("c")
    sid = lax.axis_index("s")
    iota = lax.iota(jnp.int32, 16)
    pltpu.sync_copy(cnts_hbm.at[cid, sid], cnts_v)
    cnts = cnts_v[...]
    for k in range(KPC):
        base = (2 * k + cid) * CH
        pltpu.sync_copy(temp_hbm.at[pl.ds(base + sid * STRIPE, STRIPE)],
                        shared.at[pl.ds(sid * STRIPE, STRIPE)])
        cnt = jnp.sum(jnp.where(iota == k, cnts, 0))
        nblk2 = ((cnt + 255) >> 8) * 2
        nsup = (nblk2 + 15) >> 4
        plsc.subcore_barrier()

        def sup_body(s, c, _k=k):
            pltpu.sync_copy(eids_hbm.at[cid, sid, _k, pl.ds(s * 16, 16)],
                            eids_s)
            pltpu.sync_copy(lidx_hbm.at[cid, sid, _k, pl.ds(s * 16, 16)],
                            lidx_s)
            rb = jnp.minimum(16, nblk2 - s * 16)
            pltpu.async_copy(msg_hbm.at[eids_s.at[0]], rows_a, gsem_a)

            def pair_body(q, c2):
                r0 = 2 * q
                pltpu.make_async_copy(msg_hbm.at[eids_s.at[r0]], rows_a,
                                      gsem_a).wait()
                pltpu.async_copy(msg_hbm.at[eids_s.at[r0 + 1]], rows_b,
                                 gsem_b)
                pltpu.async_copy(rows_a, shared.at[lidx_s.at[r0]], asem_a,
                                 add=True)
                pltpu.make_async_copy(msg_hbm.at[eids_s.at[r0 + 1]], rows_b,
                                      gsem_b).wait()
                pltpu.make_async_copy(rows_a, shared.at[lidx_s.at[r0]],
                                      asem_a).wait()

                @pl.when(r0 + 2 < rb)
                def _():
                    pltpu.async_copy(msg_hbm.at[eids_s.at[r0 + 2]], rows_a,
                                     gsem_a)

                pltpu.async_copy(rows_b, shared.at[lidx_s.at[r0 + 1]],
                                 asem_b, add=True)
                pltpu.make_async_copy(rows_b, shared.at[lidx_s.at[r0 + 1]],
                                      asem_b).wait()
                return c2

            lax.fori_loop(0, rb >> 1, pair_body, 0)
            return c

        lax.fori_loop(0, nsup, sup_body, 0)
        plsc.subcore_barrier()
        pltpu.sync_copy(shared.at[pl.ds(sid * STRIPE, STRIPE)],
                        out_hbm.at[pl.ds(base + sid * STRIPE, STRIPE)])
        plsc.subcore_barrier()


def kernel(feat, W_ctr, W_rel, gn1_w, gn1_b, W_ctr2, gn2_w, gn2_b,
           u_idx, v_idx):
    feat_p = jnp.concatenate(
        [feat, jnp.zeros((NP - N, D), jnp.float32)], axis=0)
    u_flat = jnp.concatenate(
        [u_idx.reshape(-1), jnp.full((TOTP - RE,), _BIG, jnp.int32)])
    v2d = jnp.concatenate(
        [v_idx.reshape(-1), jnp.zeros((TOTP - RE,), jnp.int32)]
    ).reshape(NW, GBLK, GB)

    eids, lidx, cnts = _sc_build(u_flat)

    res = feat_p
    temp = _tc_mm(feat_p, W_ctr[0])
    for i in range(L):
        gath = _sc_gather(feat_p, v2d)
        msg = _tc_msg_mm(gath, W_rel[i])
        temp = _sc_scatter(temp, msg, eids, lidx, cnts)
        gargs = (gn1_w[i].reshape(1, D), gn1_b[i].reshape(1, D),
                 W_ctr2[i], gn2_w[i].reshape(1, D), gn2_b[i].reshape(1, D))
        if i + 1 < L:
            feat_p, temp = _tc_post_mm(temp, res, *gargs, W_ctr[i + 1])
        else:
            feat_p = _tc_post(temp, res, *gargs)
        res = feat_p
    return feat_p[:N]


# speedup vs baseline: 1.2340x; 1.2340x over previous
import functools

import jax
import jax.numpy as jnp
from jax import lax
from jax.experimental import pallas as pl
from jax.experimental.pallas import tpu as pltpu
from jax.experimental.pallas import tpu_sc as plsc

N = 100000
D = 128
R = 14
E = 40000
L = 4
RE = R * E

NC, NS = 2, 16
NW = NC * NS

NP = 102400
BN = 2048
BE = 2000

GB = 128
TOTP = 561152
GBLK = TOTP // NW // GB

ET = TOTP // NS
UB = 2192
NUB = ET // UB
LROWS = 288

NCH = 10
KPC = NCH // NC
CH = NP // NCH
STRIPE = CH // NS
TRASH = CH
SPR = CH + 24

_BIG = 1 << 30

_mesh = plsc.VectorSubcoreMesh(core_axis_name="c", subcore_axis_name="s")


def _mm_body(x_ref, w_ref, o_ref):
    o_ref[...] = lax.dot_general(
        x_ref[...], w_ref[...], (((1,), (1,)), ((), ())),
        preferred_element_type=jnp.float32)


def _tc_mm(x, w):
    grid = (NP // BN,)
    return pl.pallas_call(
        _mm_body,
        grid=grid,
        in_specs=[
            pl.BlockSpec((BN, D), lambda i: (i, 0)),
            pl.BlockSpec((D, D), lambda i: (0, 0)),
        ],
        out_specs=pl.BlockSpec((BN, D), lambda i: (i, 0)),
        out_shape=jax.ShapeDtypeStruct((NP, D), jnp.float32),
    )(x, w)


def _msg_body(x_ref, w_ref, o_ref):
    o_ref[...] = lax.dot_general(
        x_ref[...], w_ref[0], (((1,), (1,)), ((), ())),
        preferred_element_type=jnp.float32)


def _tc_msg_mm(gath, w_rel_i):
    nb = E // BE
    grid = (R, nb)
    return pl.pallas_call(
        _msg_body,
        grid=grid,
        in_specs=[
            pl.BlockSpec((BE, D), lambda r, j: (r * nb + j, 0)),
            pl.BlockSpec((1, D, D), lambda r, j: (r, 0, 0)),
        ],
        out_specs=pl.BlockSpec((BE, D), lambda r, j: (r * nb + j, 0)),
        out_shape=jax.ShapeDtypeStruct((RE, D), jnp.float32),
    )(gath, w_rel_i)


def _post_body(t_ref, res_ref, g1w_ref, g1b_ref, w2_ref, g2w_ref, g2b_ref,
               o_ref):
    x = t_ref[...]
    mu = jnp.mean(x, axis=1, keepdims=True)
    var = jnp.mean((x - mu) ** 2, axis=1, keepdims=True)
    x = (x - mu) * lax.rsqrt(var + 1e-5) * g1w_ref[...] + g1b_ref[...]
    x = jnp.maximum(x, 0.0)
    y = lax.dot_general(x, w2_ref[...], (((1,), (1,)), ((), ())),
                        preferred_element_type=jnp.float32)
    mu2 = jnp.mean(y, axis=1, keepdims=True)
    var2 = jnp.mean((y - mu2) ** 2, axis=1, keepdims=True)
    y = (y - mu2) * lax.rsqrt(var2 + 1e-5) * g2w_ref[...] + g2b_ref[...]
    o_ref[...] = jnp.maximum(y + res_ref[...], 0.0)


def _tc_post(temp, res, g1w, g1b, w2, g2w, g2b):
    grid = (NP // BN,)
    vec = lambda i: (0, 0)
    return pl.pallas_call(
        _post_body,
        grid=grid,
        in_specs=[
            pl.BlockSpec((BN, D), lambda i: (i, 0)),
            pl.BlockSpec((BN, D), lambda i: (i, 0)),
            pl.BlockSpec((1, D), vec),
            pl.BlockSpec((1, D), vec),
            pl.BlockSpec((D, D), vec),
            pl.BlockSpec((1, D), vec),
            pl.BlockSpec((1, D), vec),
        ],
        out_specs=pl.BlockSpec((BN, D), lambda i: (i, 0)),
        out_shape=jax.ShapeDtypeStruct((NP, D), jnp.float32),
    )(temp, res, g1w, g1b, w2, g2w, g2b)


def _post_mm_body(t_ref, res_ref, g1w_ref, g1b_ref, w2_ref, g2w_ref,
                  g2b_ref, wn_ref, o_ref, o2_ref):
    _post_body(t_ref, res_ref, g1w_ref, g1b_ref, w2_ref, g2w_ref, g2b_ref,
               o_ref)
    o2_ref[...] = lax.dot_general(
        o_ref[...], wn_ref[...], (((1,), (1,)), ((), ())),
        preferred_element_type=jnp.float32)


def _tc_post_mm(temp, res, g1w, g1b, w2, g2w, g2b, w_next):
    grid = (NP // BN,)
    vec = lambda i: (0, 0)
    return pl.pallas_call(
        _post_mm_body,
        grid=grid,
        in_specs=[
            pl.BlockSpec((BN, D), lambda i: (i, 0)),
            pl.BlockSpec((BN, D), lambda i: (i, 0)),
            pl.BlockSpec((1, D), vec),
            pl.BlockSpec((1, D), vec),
            pl.BlockSpec((D, D), vec),
            pl.BlockSpec((1, D), vec),
            pl.BlockSpec((1, D), vec),
            pl.BlockSpec((D, D), vec),
        ],
        out_specs=[
            pl.BlockSpec((BN, D), lambda i: (i, 0)),
            pl.BlockSpec((BN, D), lambda i: (i, 0)),
        ],
        out_shape=[
            jax.ShapeDtypeStruct((NP, D), jnp.float32),
            jax.ShapeDtypeStruct((NP, D), jnp.float32),
        ],
    )(temp, res, g1w, g1b, w2, g2w, g2b, w_next)


@functools.partial(
    pl.kernel,
    out_type=jax.ShapeDtypeStruct((TOTP, D), jnp.float32),
    mesh=_mesh,
    scratch_types=[
        pltpu.VMEM((GBLK, GB), jnp.int32),
        pltpu.VMEM((GB, D), jnp.float32),
        pltpu.VMEM((GB, D), jnp.float32),
        pltpu.SemaphoreType.DMA,
        pltpu.SemaphoreType.DMA,
    ],
)
def _sc_gather(feat_hbm, v2d_hbm, out_hbm, idx_v, rows_a, rows_b,
               sem_a, sem_b):
    cid = lax.axis_index("c")
    sid = lax.axis_index("s")
    wid = sid * NC + cid
    pltpu.sync_copy(v2d_hbm.at[wid], idx_v)
    base_row = wid * GBLK * GB

    pltpu.async_copy(feat_hbm.at[idx_v.at[0]], rows_a, sem_a)

    def pair(q, c):
        b0 = 2 * q
        pltpu.async_copy(feat_hbm.at[idx_v.at[b0 + 1]], rows_b, sem_b)
        pltpu.make_async_copy(feat_hbm.at[idx_v.at[b0]], rows_a, sem_a).wait()
        pltpu.sync_copy(rows_a, out_hbm.at[pl.ds(base_row + b0 * GB, GB)])
        pltpu.async_copy(feat_hbm.at[idx_v.at[b0 + 2]], rows_a, sem_a)
        pltpu.make_async_copy(feat_hbm.at[idx_v.at[b0 + 1]], rows_b,
                              sem_b).wait()
        pltpu.sync_copy(rows_b,
                        out_hbm.at[pl.ds(base_row + (b0 + 1) * GB, GB)])
        return c

    lax.fori_loop(0, (GBLK - 1) // 2, pair, 0)
    pltpu.make_async_copy(feat_hbm.at[idx_v.at[GBLK - 1]], rows_a,
                          sem_a).wait()
    pltpu.sync_copy(rows_a,
                    out_hbm.at[pl.ds(base_row + (GBLK - 1) * GB, GB)])


@functools.partial(
    pl.kernel,
    out_type=(
        jax.ShapeDtypeStruct((NC, NS, KPC, LROWS, GB), jnp.int32),
        jax.ShapeDtypeStruct((NC, NS, KPC, LROWS, GB), jnp.int32),
        jax.ShapeDtypeStruct((NC, NS, 16), jnp.int32),
    ),
    mesh=_mesh,
    scratch_types=[
        pltpu.VMEM((UB,), jnp.int32),
        pltpu.VMEM((LROWS, GB), jnp.int32),
        pltpu.VMEM((LROWS, GB), jnp.int32),
        pltpu.VMEM((16,), jnp.int32),
    ],
    compiler_params=pltpu.CompilerParams(needs_layout_passes=False),
)
def _sc_build(u_hbm, eids_hbm, lidx_hbm, cnts_hbm, ubuf, eids_v, lidx_v,
              cnts_v):
    cid = lax.axis_index("c")
    sid = lax.axis_index("s")
    e0 = sid * ET
    iota = lax.iota(jnp.int32, 16)
    cnts = jnp.zeros((16,), jnp.int32)
    for k in range(KPC):
        base = (2 * k + cid) * CH
        cnt = jnp.int32(0)
        for ub in range(NUB):
            pltpu.sync_copy(u_hbm.at[pl.ds(e0 + ub * UB, UB)], ubuf)

            def scan_body(j, cnt, _ub=ub):
                u16 = ubuf[pl.ds(j * 16, 16)]
                m = (u16 >= base) & (u16 < base + CH)
                mi = jnp.where(m, 1, 0)
                pos = cnt + plsc.cumsum(mi) - 1
                eid16 = e0 + _ub * UB + j * 16 + iota
                plsc.store_scatter(eids_v, [pos >> 7, pos & 127], eid16,
                                   mask=m)
                plsc.store_scatter(lidx_v, [pos >> 7, pos & 127], u16 - base,
                                   mask=m)
                return cnt + jnp.sum(mi)

            cnt = lax.fori_loop(0, UB // 16, scan_body, cnt)
        for pb in range(2 * GB // 16):
            pos = cnt + pb * 16 + iota
            plsc.store_scatter(eids_v, [pos >> 7, pos & 127],
                               jnp.zeros((16,), jnp.int32))
            plsc.store_scatter(lidx_v, [pos >> 7, pos & 127],
                               TRASH + iota)
        pltpu.sync_copy(eids_v, eids_hbm.at[cid, sid, k])
        pltpu.sync_copy(lidx_v, lidx_hbm.at[cid, sid, k])
        cnts = jnp.where(iota == k, cnt, cnts)
    cnts_v[...] = cnts
    pltpu.sync_copy(cnts_v, cnts_hbm.at[cid, sid])


@functools.partial(
    pl.kernel,
    out_type=jax.ShapeDtypeStruct((NP, D), jnp.float32),
    mesh=_mesh,
    scratch_types=[
        pltpu.VMEM_SHARED((SPR, D), jnp.float32),
        pltpu.VMEM((16, GB), jnp.int32),
        pltpu.VMEM((16, GB), jnp.int32),
        pltpu.VMEM((GB, D), jnp.float32),
        pltpu.VMEM((GB, D), jnp.float32),
        pltpu.VMEM((16,), jnp.int32),
        pltpu.SemaphoreType.DMA,
        pltpu.SemaphoreType.DMA,
    ],
    compiler_params=pltpu.CompilerParams(needs_layout_passes=False),
)
def _sc_scatter(temp_hbm, msg_hbm, eids_hbm, lidx_hbm, cnts_hbm, out_hbm,
                shared, eids_s, lidx_s, rows_a, rows_b, cnts_v,
                sem_a, sem_b):
    cid = lax.axis_index("c")
    sid = lax.axis_index("s")
    iota = lax.iota(jnp.int32, 16)
    pltpu.sync_copy(cnts_hbm.at[cid, sid], cnts_v)
    cnts = cnts_v[...]
    for k in range(KPC):
        base = (2 * k + cid) * CH
        pltpu.sync_copy(temp_hbm.at[pl.ds(base + sid * STRIPE, STRIPE)],
                        shared.at[pl.ds(sid * STRIPE, STRIPE)])
        cnt = jnp.sum(jnp.where(iota == k, cnts, 0))
        nblk2 = (cnt + GB - 1) >> 7
        nsup = (nblk2 + 15) >> 4
        plsc.subcore_barrier()

        def sup_body(s, c, _k=k):
            pltpu.sync_copy(eids_hbm.at[cid, sid, _k, pl.ds(s * 16, 16)],
                            eids_s)
            pltpu.sync_copy(lidx_hbm.at[cid, sid, _k, pl.ds(s * 16, 16)],
                            lidx_s)
            rb = jnp.minimum(16, nblk2 - s * 16)

            def blk_body(rr, c2):
                pltpu.async_copy(msg_hbm.at[eids_s.at[rr]], rows_a,
                                 sem_a).wait()
                pltpu.sync_copy(rows_a, shared.at[lidx_s.at[rr]], add=True)
                return c2

            lax.fori_loop(0, rb, blk_body, 0)
            return c

        lax.fori_loop(0, nsup, sup_body, 0)
        plsc.subcore_barrier()
        pltpu.sync_copy(shared.at[pl.ds(sid * STRIPE, STRIPE)],
                        out_hbm.at[pl.ds(base + sid * STRIPE, STRIPE)])
        plsc.subcore_barrier()


def kernel(feat, W_ctr, W_rel, gn1_w, gn1_b, W_ctr2, gn2_w, gn2_b,
           u_idx, v_idx):
    feat_p = jnp.concatenate(
        [feat, jnp.zeros((NP - N, D), jnp.float32)], axis=0)
    u_flat = jnp.concatenate(
        [u_idx.reshape(-1), jnp.full((TOTP - RE,), _BIG, jnp.int32)])
    v2d = jnp.concatenate(
        [v_idx.reshape(-1), jnp.zeros((TOTP - RE,), jnp.int32)]
    ).reshape(NW, GBLK, GB)

    eids, lidx, cnts = _sc_build(u_flat)

    res = feat_p
    temp = _tc_mm(feat_p, W_ctr[0])
    for i in range(L):
        gath = _sc_gather(feat_p, v2d)
        msg = _tc_msg_mm(gath, W_rel[i])
        temp = _sc_scatter(temp, msg, eids, lidx, cnts)
        gargs = (gn1_w[i].reshape(1, D), gn1_b[i].reshape(1, D),
                 W_ctr2[i], gn2_w[i].reshape(1, D), gn2_b[i].reshape(1, D))
        if i + 1 < L:
            feat_p, temp = _tc_post_mm(temp, res, *gargs, W_ctr[i + 1])
        else:
            feat_p = _tc_post(temp, res, *gargs)
        res = feat_p
    return feat_p[:N]
